# baseline (speedup 1.0000x reference)
import jax
import jax.numpy as jnp
from jax import lax
from jax.experimental import pallas as pl
from jax.experimental.pallas import tpu as pltpu
from jax.experimental.pallas import tpu_sc as plsc

_EMBED = 64
_NC = 2
_NS = 16
_NW = _NC * _NS
_NBUF = 4


def _sc_gather(idx_hbm, table_hbm, out_hbm, idx_v, rows, gsem, wsem):
    cpw = idx_hbm.shape[1]
    half = idx_hbm.shape[3]
    wid = lax.axis_index("s") * _NC + lax.axis_index("c")
    rbase = wid * cpw
    pltpu.sync_copy(idx_hbm.at[wid], idx_v)

    def gdesc(j, b, h):
        return pltpu.make_async_copy(
            table_hbm.at[idx_v.at[j, h]], rows.at[b, h], gsem.at[b])

    def gstart(j, b):
        gdesc(j, b, 0).start()
        gdesc(j, b, 1).start()

    def gwait(j, b):
        gdesc(j, b, 0).wait()
        gdesc(j, b, 1).wait()

    def wdesc(j, b, h):
        return pltpu.make_async_copy(
            rows.at[b, h],
            out_hbm.at[pl.ds((rbase + j) * half, half),
                       pl.ds(h * _EMBED, _EMBED)],
            wsem.at[b],
        )

    def wstart(j, b):
        wdesc(j, b, 0).start()
        wdesc(j, b, 1).start()

    def wwait(j, b):
        wdesc(j, b, 0).wait()
        wdesc(j, b, 1).wait()

    for b in range(_NBUF):
        gstart(b, b)

    nsteps = cpw // _NBUF

    def body(step, carry):
        base = step * _NBUF
        for b in range(_NBUF):
            gwait(base + b, b)
            wstart(base + b, b)
        for b in range(_NBUF):
            wwait(base + b, b)
            gstart(base + _NBUF + b, b)
        return carry

    lax.fori_loop(0, nsteps - 1, body, 0)
    tail = (nsteps - 1) * _NBUF
    for b in range(_NBUF):
        gwait(tail + b, b)
        wstart(tail + b, b)
    for b in range(_NBUF):
        wwait(tail + b, b)


def kernel(x, encoding):
    n, s = x.shape
    cpw = n // _NW
    half = s // 2
    xr = x.reshape(_NW, cpw, s).astype(jnp.int32)
    idx = jnp.stack([xr[:, :, 0::2], xr[:, :, 1::2]], axis=2)
    out = pl.kernel(
        _sc_gather,
        out_type=jax.ShapeDtypeStruct((n * half, 2 * _EMBED), jnp.float32),
        mesh=plsc.VectorSubcoreMesh(core_axis_name="c", subcore_axis_name="s"),
        compiler_params=pltpu.CompilerParams(use_tc_tiling_on_sc=False),
        scratch_types=[
            pltpu.VMEM((cpw, 2, half), jnp.int32),
            pltpu.VMEM((_NBUF, 2, half, _EMBED), jnp.float32),
            pltpu.SemaphoreType.DMA((_NBUF,)),
            pltpu.SemaphoreType.DMA((_NBUF,)),
        ],
    )(idx, encoding)
    return out.reshape(n, s, _EMBED)
